# baseline (speedup 1.0000x reference)
import functools

import jax
import jax.numpy as jnp
from jax import lax
from jax.experimental import pallas as pl
from jax.experimental.pallas import tpu as pltpu
from jax.experimental.pallas import tpu_sc as plsc

N = 10000
D = 128
E = 320000

NC = 2
NS = 16
L = 16

NW = NC * NS
E_PER_TILE = E // NW
CH = 80
ET_PAD = 10000
NCHUNK = ET_PAD // CH
NSLAB = 5
CS = NCHUNK // NSLAB
NP = 10240
SLAB = NP // NS


SLABW = CS * CH


def _sc_aggregate(x, ei_f, ei_l, ei_v, z2d, z1d):
    mesh = plsc.VectorSubcoreMesh(core_axis_name="c", subcore_axis_name="s")

    @functools.partial(
        pl.kernel,
        mesh=mesh,
        out_type=[
            jax.ShapeDtypeStruct((3, NC, NP, D), jnp.float32),
            jax.ShapeDtypeStruct((3, NC, NP), jnp.float32),
        ],
        scratch_types=[
            pltpu.VMEM((2 * SLABW,), jnp.int32),
            pltpu.VMEM((2 * SLABW,), jnp.int32),
            pltpu.VMEM((CH, D), jnp.float32),
            pltpu.VMEM((CH, D), jnp.float32),
            pltpu.VMEM((CH,), jnp.float32),
            pltpu.VMEM_SHARED((NP, D), jnp.float32),
            pltpu.VMEM_SHARED((NP,), jnp.float32),
            pltpu.SemaphoreType.DMA,
            pltpu.SemaphoreType.DMA,
            pltpu.SemaphoreType.DMA,
            pltpu.SemaphoreType.DMA,
            pltpu.SemaphoreType.DMA,
        ],
    )
    def k(x_hbm, e0, e1, e2, z2d_hbm, z1d_hbm, sums_out, cnts_out,
          slabA, slabB, rows0, rows1, ones_v, acc_sh, cnt_sh,
          sem0, sem1, sems, semc, semL):
        c = lax.axis_index("c")
        s = lax.axis_index("s")
        wid = s * NC + c

        for kk in range(CH // L):
            ones_v[pl.ds(kk * L, L)] = jnp.ones((L,), jnp.float32)

        def fire(sl, j, buf, sm):
            pltpu.async_copy(x_hbm.at[sl.at[pl.ds(j * CH, CH)]], buf, sm)

        def wait_gather(buf, sm):
            pltpu.make_async_copy(x_hbm.at[slabA.at[pl.ds(0, CH)]], buf, sm).wait()

        def load_slab(ee, si, sl, sync):
            off = pl.multiple_of(wid * E_PER_TILE + si * SLABW, 8)
            if sync:
                pltpu.sync_copy(ee.at[pl.ds(off, SLABW)], sl.at[pl.ds(0, SLABW)])
                pltpu.sync_copy(ee.at[pl.ds(E + off, SLABW)],
                                sl.at[pl.ds(SLABW, SLABW)])
            else:
                pltpu.async_copy(ee.at[pl.ds(off, SLABW)],
                                 sl.at[pl.ds(0, SLABW)], semL)
                pltpu.async_copy(ee.at[pl.ds(E + off, SLABW)],
                                 sl.at[pl.ds(SLABW, SLABW)], semL)

        def wait_slab(ee, sl):
            pltpu.make_async_copy(ee.at[pl.ds(0, SLABW)],
                                  sl.at[pl.ds(0, SLABW)], semL).wait()
            pltpu.make_async_copy(ee.at[pl.ds(0, SLABW)],
                                  sl.at[pl.ds(SLABW, SLABW)], semL).wait()

        def step(sl, j, buf, smg, do_fire=True):
            wait_gather(buf, smg)
            didx = sl.at[pl.ds(SLABW + j * CH, CH)]
            pltpu.async_copy(buf, acc_sh.at[didx], sems, add=True)
            pltpu.async_copy(ones_v, cnt_sh.at[didx], semc, add=True)
            pltpu.make_async_copy(buf, acc_sh.at[sl.at[pl.ds(SLABW, CH)]],
                                  sems).wait()
            if do_fire:
                @pl.when(j + 2 < CS)
                def _():
                    fire(sl, j + 2, buf, smg)

        bufs = (rows0, rows1)
        gsems = (sem0, sem1)

        for et, ee in enumerate((e0, e1, e2)):
            load_slab(ee, 0, slabA, sync=True)
            fire(slabA, 0, rows0, sem0)
            fire(slabA, 1, rows1, sem1)
            pltpu.sync_copy(z2d_hbm.at[pl.ds(s * SLAB, SLAB)],
                            acc_sh.at[pl.ds(s * SLAB, SLAB)])

            @pl.when(s == 0)
            def _():
                pltpu.sync_copy(z1d_hbm, cnt_sh)

            plsc.subcore_barrier()

            for si in range(NSLAB):
                p = si % 2
                cur = slabA if p == 0 else slabB
                nxt = slabB if p == 0 else slabA
                b0, b1 = bufs[p], bufs[1 - p]
                g0, g1 = gsems[p], gsems[1 - p]
                if si + 1 < NSLAB:
                    load_slab(ee, si + 1, nxt, sync=False)

                def outer(i, carry, cur=cur, b0=b0, b1=b1, g0=g0, g1=g1):
                    jj = i * 2
                    step(cur, jj, b0, g0)
                    step(cur, jj + 1, b1, g1)
                    return carry

                lax.fori_loop(0, (CS - 1) // 2, outer, 0)
                step(cur, CS - 1, b0, g0, do_fire=False)

                def drain_cnt(i, carry, cur=cur):
                    pltpu.make_async_copy(
                        ones_v, cnt_sh.at[cur.at[pl.ds(SLABW, CH)]], semc).wait()
                    return carry

                lax.fori_loop(0, CS, drain_cnt, 0)
                if si + 1 < NSLAB:
                    wait_slab(ee, nxt)
                    fire(nxt, 0, b1, g1)
                    fire(nxt, 1, b0, g0)

            plsc.subcore_barrier()

            pltpu.sync_copy(acc_sh.at[pl.ds(s * SLAB, SLAB)],
                            sums_out.at[et, c, pl.ds(s * SLAB, SLAB)])

            @pl.when(s == 0)
            def _():
                pltpu.sync_copy(cnt_sh, cnts_out.at[et, c])

            plsc.subcore_barrier()

    return k(x, ei_f, ei_l, ei_v, z2d, z1d)


R = 1000


def _tc_finalize_body(sums_ref, cnts_ref, wt_ref, b_ref, out_ref):
    acc = jnp.zeros((R, D), jnp.float32)
    for et in range(3):
        sres = sums_ref[et, 0] + sums_ref[et, 1]
        cres = cnts_ref[et, 0] + cnts_ref[et, 1]
        inv = 1.0 / jnp.maximum(cres, 1.0)
        acc = acc + jnp.dot(sres * inv, wt_ref[et], preferred_element_type=jnp.float32)
        acc = acc + jnp.where(cres > 0.0, 1.0, 0.0) * b_ref[et]
    out_ref[...] = acc


def _tc_finalize(sums, cnts, wt, b):
    return pl.pallas_call(
        _tc_finalize_body,
        grid=(N // R,),
        in_specs=[
            pl.BlockSpec((3, NC, R, D), lambda i: (0, 0, i, 0)),
            pl.BlockSpec((3, NC, R, 1), lambda i: (0, 0, i, 0)),
            pl.BlockSpec((3, D, D), lambda i: (0, 0, 0)),
            pl.BlockSpec((3, 1, D), lambda i: (0, 0, 0)),
        ],
        out_specs=pl.BlockSpec((R, D), lambda i: (i, 0)),
        out_shape=jax.ShapeDtypeStruct((N, D), jnp.float32),
    )(sums, cnts, wt, b)


def kernel(x, edge_index_follows, edge_index_likes, edge_index_views,
           W_follows, b_follows, W_likes, b_likes, W_views, b_views):
    z2d = jnp.zeros((NP, D), jnp.float32)
    z1d = jnp.zeros((NP,), jnp.float32)
    sums, cnts = _sc_aggregate(x, edge_index_follows.reshape(-1),
                               edge_index_likes.reshape(-1),
                               edge_index_views.reshape(-1), z2d, z1d)
    wt = jnp.stack([W_follows.T, W_likes.T, W_views.T])
    b = jnp.stack([b_follows, b_likes, b_views])[:, None, :]
    return _tc_finalize(sums, cnts.reshape(3, NC, NP, 1), wt, b)


# speedup vs baseline: 1.0315x; 1.0315x over previous
import functools

import jax
import jax.numpy as jnp
from jax import lax
from jax.experimental import pallas as pl
from jax.experimental.pallas import tpu as pltpu
from jax.experimental.pallas import tpu_sc as plsc

N = 10000
D = 128
E = 320000

NC = 2
NS = 16
L = 16

NW = NC * NS
E_PER_TILE = E // NW
CH = 80
ET_PAD = 10000
NCHUNK = ET_PAD // CH
NSLAB = 5
CS = NCHUNK // NSLAB
NP = 10240
SLAB = NP // NS


SLABW = CS * CH


def _sc_aggregate(x, ei_f, ei_l, ei_v, z1d):
    mesh = plsc.VectorSubcoreMesh(core_axis_name="c", subcore_axis_name="s")

    @functools.partial(
        pl.kernel,
        mesh=mesh,
        out_type=[
            jax.ShapeDtypeStruct((3, NC, NP, D), jnp.float32),
            jax.ShapeDtypeStruct((3, NC, NP), jnp.float32),
        ],
        scratch_types=[
            pltpu.VMEM((2 * SLABW,), jnp.int32),
            pltpu.VMEM((2 * SLABW,), jnp.int32),
            pltpu.VMEM((CH, D), jnp.float32),
            pltpu.VMEM((CH, D), jnp.float32),
            pltpu.VMEM((CH,), jnp.float32),
            pltpu.VMEM((CH, D), jnp.float32),
            pltpu.VMEM_SHARED((NP, D), jnp.float32),
            pltpu.VMEM_SHARED((NP,), jnp.float32),
            pltpu.SemaphoreType.DMA,
            pltpu.SemaphoreType.DMA,
            pltpu.SemaphoreType.DMA,
            pltpu.SemaphoreType.DMA,
            pltpu.SemaphoreType.DMA,
        ],
    )
    def k(x_hbm, e0, e1, e2, z1d_hbm, sums_out, cnts_out,
          slabA, slabB, rows0, rows1, ones_v, zero_v, acc_sh, cnt_sh,
          sem0, sem1, sems, semc, semL):
        c = lax.axis_index("c")
        s = lax.axis_index("s")
        wid = s * NC + c

        for kk in range(CH // L):
            ones_v[pl.ds(kk * L, L)] = jnp.ones((L,), jnp.float32)

        def zinit(i, carry):
            zero_v[i, pl.ds(0, L)] = jnp.zeros((L,), jnp.float32)
            for kk in range(1, D // L):
                zero_v[i, pl.ds(kk * L, L)] = jnp.zeros((L,), jnp.float32)
            return carry

        lax.fori_loop(0, CH, zinit, 0)

        def fire(sl, j, buf, sm):
            pltpu.async_copy(x_hbm.at[sl.at[pl.ds(j * CH, CH)]], buf, sm)

        def wait_gather(buf, sm):
            pltpu.make_async_copy(x_hbm.at[slabA.at[pl.ds(0, CH)]], buf, sm).wait()

        def load_slab(ee, si, sl, sync):
            off = pl.multiple_of(wid * E_PER_TILE + si * SLABW, 8)
            if sync:
                pltpu.sync_copy(ee.at[pl.ds(off, SLABW)], sl.at[pl.ds(0, SLABW)])
                pltpu.sync_copy(ee.at[pl.ds(E + off, SLABW)],
                                sl.at[pl.ds(SLABW, SLABW)])
            else:
                pltpu.async_copy(ee.at[pl.ds(off, SLABW)],
                                 sl.at[pl.ds(0, SLABW)], semL)
                pltpu.async_copy(ee.at[pl.ds(E + off, SLABW)],
                                 sl.at[pl.ds(SLABW, SLABW)], semL)

        def wait_slab(ee, sl):
            pltpu.make_async_copy(ee.at[pl.ds(0, SLABW)],
                                  sl.at[pl.ds(0, SLABW)], semL).wait()
            pltpu.make_async_copy(ee.at[pl.ds(0, SLABW)],
                                  sl.at[pl.ds(SLABW, SLABW)], semL).wait()

        def step(sl, j, buf, smg, do_fire=True):
            wait_gather(buf, smg)
            didx = sl.at[pl.ds(SLABW + j * CH, CH)]
            pltpu.async_copy(buf, acc_sh.at[didx], sems, add=True)
            pltpu.async_copy(ones_v, cnt_sh.at[didx], semc, add=True)
            pltpu.make_async_copy(buf, acc_sh.at[sl.at[pl.ds(SLABW, CH)]],
                                  sems).wait()
            if do_fire:
                @pl.when(j + 2 < CS)
                def _():
                    fire(sl, j + 2, buf, smg)

        bufs = (rows0, rows1)
        gsems = (sem0, sem1)

        for et, ee in enumerate((e0, e1, e2)):
            load_slab(ee, 0, slabA, sync=True)
            fire(slabA, 0, rows0, sem0)
            fire(slabA, 1, rows1, sem1)
            for zz in range(SLAB // CH):
                pltpu.sync_copy(
                    zero_v, acc_sh.at[pl.ds(s * SLAB + zz * CH, CH)])

            @pl.when(s == 0)
            def _():
                pltpu.sync_copy(z1d_hbm, cnt_sh)

            plsc.subcore_barrier()

            for si in range(NSLAB):
                p = si % 2
                cur = slabA if p == 0 else slabB
                nxt = slabB if p == 0 else slabA
                b0, b1 = bufs[p], bufs[1 - p]
                g0, g1 = gsems[p], gsems[1 - p]
                if si + 1 < NSLAB:
                    load_slab(ee, si + 1, nxt, sync=False)

                def outer(i, carry, cur=cur, b0=b0, b1=b1, g0=g0, g1=g1):
                    jj = i * 2
                    step(cur, jj, b0, g0)
                    step(cur, jj + 1, b1, g1)
                    return carry

                lax.fori_loop(0, (CS - 1) // 2, outer, 0)
                step(cur, CS - 1, b0, g0, do_fire=False)

                def drain_cnt(i, carry, cur=cur):
                    pltpu.make_async_copy(
                        ones_v, cnt_sh.at[cur.at[pl.ds(SLABW, CH)]], semc).wait()
                    return carry

                lax.fori_loop(0, CS, drain_cnt, 0)
                if si + 1 < NSLAB:
                    wait_slab(ee, nxt)
                    fire(nxt, 0, b1, g1)
                    fire(nxt, 1, b0, g0)

            plsc.subcore_barrier()

            pltpu.sync_copy(acc_sh.at[pl.ds(s * SLAB, SLAB)],
                            sums_out.at[et, c, pl.ds(s * SLAB, SLAB)])

            @pl.when(s == 0)
            def _():
                pltpu.sync_copy(cnt_sh, cnts_out.at[et, c])

            plsc.subcore_barrier()

    return k(x, ei_f, ei_l, ei_v, z1d)


R = 1000


def _tc_finalize_body(sums_ref, cnts_ref, wt_ref, b_ref, out_ref):
    acc = jnp.zeros((R, D), jnp.float32)
    for et in range(3):
        sres = sums_ref[et, 0] + sums_ref[et, 1]
        cres = cnts_ref[et, 0] + cnts_ref[et, 1]
        inv = 1.0 / jnp.maximum(cres, 1.0)
        acc = acc + jnp.dot(sres * inv, wt_ref[et], preferred_element_type=jnp.float32)
        acc = acc + jnp.where(cres > 0.0, 1.0, 0.0) * b_ref[et]
    out_ref[...] = acc


def _tc_finalize(sums, cnts, wt, b):
    return pl.pallas_call(
        _tc_finalize_body,
        grid=(N // R,),
        in_specs=[
            pl.BlockSpec((3, NC, R, D), lambda i: (0, 0, i, 0)),
            pl.BlockSpec((3, NC, R, 1), lambda i: (0, 0, i, 0)),
            pl.BlockSpec((3, D, D), lambda i: (0, 0, 0)),
            pl.BlockSpec((3, 1, D), lambda i: (0, 0, 0)),
        ],
        out_specs=pl.BlockSpec((R, D), lambda i: (i, 0)),
        out_shape=jax.ShapeDtypeStruct((N, D), jnp.float32),
    )(sums, cnts, wt, b)


def kernel(x, edge_index_follows, edge_index_likes, edge_index_views,
           W_follows, b_follows, W_likes, b_likes, W_views, b_views):
    z1d = jnp.zeros((NP,), jnp.float32)
    sums, cnts = _sc_aggregate(x, edge_index_follows.reshape(-1),
                               edge_index_likes.reshape(-1),
                               edge_index_views.reshape(-1), z1d)
    wt = jnp.stack([W_follows.T, W_likes.T, W_views.T])
    b = jnp.stack([b_follows, b_likes, b_views])[:, None, :]
    return _tc_finalize(sums, cnts.reshape(3, NC, NP, 1), wt, b)
